# baseline (speedup 1.0000x reference)
import jax
import jax.numpy as jnp
from jax import lax
from jax.experimental import pallas as pl
from jax.experimental.pallas import tpu as pltpu
from jax.experimental.pallas import tpu_sc as plsc

NC = 2
NS = 16
L = 16
NW = NC * NS
B = 16384
D = 32
NM = 100000
MB = 128
NFULL = NM // MB
TAIL = NM - NFULL * MB
BPW = B // NW
ICH = 128
NCH = BPW // ICH
NBUF = 4

_MESH = dict(core_axis_name="c", subcore_axis_name="s",
             num_cores=NC, num_subcores=NS)


def _relayout_body(ut_hbm, vt_hbm, utail_hbm, vtail_hbm, u_out, v_out,
                   blk2, stage2, in_sem, out_sem):
    c = lax.axis_index("c")
    s = lax.axis_index("s")
    wid = s * NC + c
    lanes = lax.iota(jnp.int32, L)

    def do_table(t_hbm, tail_hbm, t_out):
        nrounds = -(-NFULL // NW)

        def start_in(r):
            beta = r * NW + wid

            @pl.when(beta < NFULL)
            def _():
                pltpu.async_copy(t_hbm.at[:, pl.ds(beta * MB, MB)],
                                 blk2.at[pl.ds((r % NBUF) * D, D)], in_sem)

        for r0 in range(NBUF - 1):
            start_in(r0)

        def round_fn(r, carry):
            beta = r * NW + wid
            par = r % NBUF

            @pl.when(beta < NFULL)
            def _():
                m0 = beta * MB
                pltpu.make_async_copy(t_hbm.at[:, pl.ds(m0, MB)],
                                      blk2.at[pl.ds(par * D, D)],
                                      in_sem).wait()
                start_in(r + NBUF - 1)
                @pl.when(r >= NBUF)
                def _():
                    b2 = (r - NBUF) * NW + wid
                    pltpu.make_async_copy(
                        stage2.at[par],
                        t_out.at[pl.ds(b2 * MB * D, MB * D)],
                        out_sem).wait()
                parv = jnp.full((L,), par, jnp.int32)
                cls = []
                for c0 in range(D):
                    cl = (c0 + lanes) & (D - 1)
                    cls.append((cl, par * D + cl))
                lanes32 = lanes * D
                for rho0 in range(0, MB, L):
                    rho = rho0 + lanes
                    rho32 = rho0 * D + lanes32
                    for c0 in range(D):
                        cl, cl_off = cls[c0]
                        val = plsc.load_gather(blk2, [cl_off, rho])
                        plsc.store_scatter(stage2, [parv, rho32 + cl], val)
                pltpu.async_copy(stage2.at[par],
                                 t_out.at[pl.ds(m0 * D, MB * D)],
                                 out_sem)
            return carry

        lax.fori_loop(0, nrounds, round_fn, 0)

        for r in range(max(nrounds - NBUF - 1, 0), nrounds):
            beta = r * NW + wid

            @pl.when((beta < NFULL) & (beta + NBUF * NW >= NFULL))
            def _(r=r, beta=beta):
                pltpu.make_async_copy(
                    stage2.at[r % NBUF],
                    t_out.at[pl.ds(beta * MB * D, MB * D)],
                    out_sem).wait()

        @pl.when(wid == 0)
        def _():
            pltpu.sync_copy(tail_hbm, stage2.at[0, pl.ds(0, TAIL * D)])
            pltpu.sync_copy(stage2.at[0, pl.ds(0, TAIL * D)],
                            t_out.at[pl.ds(NFULL * MB * D, TAIL * D)])

    do_table(ut_hbm, utail_hbm, u_out)
    do_table(vt_hbm, vtail_hbm, v_out)


def _gather_body(i_hbm, j_hbm, k_hbm, u_hbm, v_hbm, out_hbm,
                 idx_i, idx_j, idx_k, rows_u, rows_vj, rows_vk, out_v, sem):
    c = lax.axis_index("c")
    s = lax.axis_index("s")
    wid = s * NC + c
    base = wid * BPW

    pltpu.sync_copy(i_hbm.at[pl.ds(base, BPW)], idx_i)
    pltpu.sync_copy(j_hbm.at[pl.ds(base, BPW)], idx_j)
    pltpu.sync_copy(k_hbm.at[pl.ds(base, BPW)], idx_k)

    copies = []
    for q in range(NCH):
        isl = pl.ds(q * ICH, ICH)
        sl = pl.ds(q * ICH, ICH)
        copies.append(pltpu.async_copy(u_hbm.at[idx_i.at[isl]], rows_u.at[sl], sem))
        copies.append(pltpu.async_copy(v_hbm.at[idx_j.at[isl]], rows_vj.at[sl], sem))
        copies.append(pltpu.async_copy(v_hbm.at[idx_k.at[isl]], rows_vk.at[sl], sem))
    for cp in copies:
        cp.wait()

    lanes = lax.iota(jnp.int32, L)

    def chunk(cidx, carry):
        rbase = cidx * L
        rows = rbase + lanes
        accj = jnp.zeros((L,), jnp.float32)
        acck = jnp.zeros((L,), jnp.float32)
        for d in range(D):
            col = (lanes + d) & (D - 1)
            u = plsc.load_gather(rows_u, [rows, col])
            vj = plsc.load_gather(rows_vj, [rows, col])
            vk = plsc.load_gather(rows_vk, [rows, col])
            dj = u - vj
            dk = u - vk
            accj = accj + dj * dj
            acck = acck + dk * dk
        x = acck - accj
        out_v[pl.ds(rbase, L)] = 1.0 / (1.0 + jnp.exp(-x))
        return carry

    lax.fori_loop(0, BPW // L, chunk, 0)
    pltpu.sync_copy(out_v, out_hbm.at[pl.ds(base, BPW)])


@jax.jit
def kernel(i, j, k, U, V):
    relayout = pl.kernel(
        _relayout_body,
        out_type=(jax.ShapeDtypeStruct((NM * D,), jnp.float32),
                  jax.ShapeDtypeStruct((NM * D,), jnp.float32)),
        mesh=plsc.VectorSubcoreMesh(**_MESH),
        scratch_types=[
            pltpu.VMEM((NBUF * D, MB), jnp.float32),
            pltpu.VMEM((NBUF, MB * D), jnp.float32),
            pltpu.SemaphoreType.DMA,
            pltpu.SemaphoreType.DMA,
        ],
        compiler_params=pltpu.CompilerParams(
            needs_layout_passes=False, use_tc_tiling_on_sc=True),
    )
    u_tail = U[NFULL * MB:, :].reshape(TAIL * D)
    v_tail = V[NFULL * MB:, :].reshape(TAIL * D)
    U_lin, V_lin = relayout(U.T, V.T, u_tail, v_tail)

    gather = pl.kernel(
        _gather_body,
        out_type=jax.ShapeDtypeStruct((B,), jnp.float32),
        mesh=plsc.VectorSubcoreMesh(**_MESH),
        scratch_types=[
            pltpu.VMEM((BPW,), jnp.int32),
            pltpu.VMEM((BPW,), jnp.int32),
            pltpu.VMEM((BPW,), jnp.int32),
            pltpu.VMEM((BPW, D), jnp.float32),
            pltpu.VMEM((BPW, D), jnp.float32),
            pltpu.VMEM((BPW, D), jnp.float32),
            pltpu.VMEM((BPW,), jnp.float32),
            pltpu.SemaphoreType.DMA,
        ],
        compiler_params=pltpu.CompilerParams(
            needs_layout_passes=False, use_tc_tiling_on_sc=False),
    )
    return gather(i, j, k, U_lin.reshape(NM, D), V_lin.reshape(NM, D))


# speedup vs baseline: 1.6190x; 1.6190x over previous
import jax
import jax.numpy as jnp
from jax import lax
from jax.experimental import pallas as pl
from jax.experimental.pallas import tpu as pltpu
from jax.experimental.pallas import tpu_sc as plsc

NC = 2
NS = 16
L = 16
NW = NC * NS
B = 16384
D = 32
NM = 100000
MB = 128
NFULL = NM // MB
TAIL = NM - NFULL * MB
BPW = B // NW
ICH = 128
NCH = BPW // ICH
NBUF = 4

_MESH = dict(core_axis_name="c", subcore_axis_name="s",
             num_cores=NC, num_subcores=NS)


def _relayout_body(ut_hbm, vt_hbm, utail_hbm, vtail_hbm, u_out, v_out,
                   blk2, stage2, in_sem, out_sem):
    c = lax.axis_index("c")
    s = lax.axis_index("s")
    wid = s * NC + c
    lanes = lax.iota(jnp.int32, L)

    def do_table(t_hbm, tail_hbm, t_out):
        nrounds = -(-NFULL // NW)

        def start_in(r):
            beta = r * NW + wid

            @pl.when(beta < NFULL)
            def _():
                pltpu.async_copy(t_hbm.at[:, pl.ds(beta * MB, MB)],
                                 blk2.at[pl.ds((r % NBUF) * D, D)], in_sem)

        for r0 in range(NBUF - 1):
            start_in(r0)

        def round_fn(r, carry):
            beta = r * NW + wid
            par = r % NBUF

            @pl.when(beta < NFULL)
            def _():
                m0 = beta * MB
                pltpu.make_async_copy(t_hbm.at[:, pl.ds(m0, MB)],
                                      blk2.at[pl.ds(par * D, D)],
                                      in_sem).wait()
                start_in(r + NBUF - 1)
                @pl.when(r >= NBUF)
                def _():
                    b2 = (r - NBUF) * NW + wid
                    pltpu.make_async_copy(
                        stage2.at[par],
                        t_out.at[pl.ds(b2 * MB * D, MB * D)],
                        out_sem).wait()
                parv = jnp.full((L,), par, jnp.int32)
                lanes32 = lanes * D

                @plsc.parallel_loop(0, MB, step=L, unroll=2)
                def _(rho0):
                    rho = rho0 + lanes
                    rho32 = rho0 * D + lanes32
                    for c0 in range(D):
                        cl = (c0 + lanes) & (D - 1)
                        val = plsc.load_gather(blk2, [par * D + cl, rho])
                        plsc.store_scatter(stage2, [parv, rho32 + cl], val)
                pltpu.async_copy(stage2.at[par],
                                 t_out.at[pl.ds(m0 * D, MB * D)],
                                 out_sem)
            return carry

        lax.fori_loop(0, nrounds, round_fn, 0)

        for r in range(max(nrounds - NBUF - 1, 0), nrounds):
            beta = r * NW + wid

            @pl.when((beta < NFULL) & (beta + NBUF * NW >= NFULL))
            def _(r=r, beta=beta):
                pltpu.make_async_copy(
                    stage2.at[r % NBUF],
                    t_out.at[pl.ds(beta * MB * D, MB * D)],
                    out_sem).wait()

        @pl.when(wid == 0)
        def _():
            pltpu.sync_copy(tail_hbm, stage2.at[0, pl.ds(0, TAIL * D)])
            pltpu.sync_copy(stage2.at[0, pl.ds(0, TAIL * D)],
                            t_out.at[pl.ds(NFULL * MB * D, TAIL * D)])

    do_table(ut_hbm, utail_hbm, u_out)
    do_table(vt_hbm, vtail_hbm, v_out)


def _gather_body(i_hbm, j_hbm, k_hbm, u_hbm, v_hbm, out_hbm,
                 idx_i, idx_j, idx_k, rows_u, rows_vj, rows_vk, out_v, sem):
    c = lax.axis_index("c")
    s = lax.axis_index("s")
    wid = s * NC + c
    base = wid * BPW

    pltpu.sync_copy(i_hbm.at[pl.ds(base, BPW)], idx_i)
    pltpu.sync_copy(j_hbm.at[pl.ds(base, BPW)], idx_j)
    pltpu.sync_copy(k_hbm.at[pl.ds(base, BPW)], idx_k)

    copies = []
    for q in range(NCH):
        isl = pl.ds(q * ICH, ICH)
        sl = pl.ds(q * ICH, ICH)
        copies.append(pltpu.async_copy(u_hbm.at[idx_i.at[isl]], rows_u.at[sl], sem))
        copies.append(pltpu.async_copy(v_hbm.at[idx_j.at[isl]], rows_vj.at[sl], sem))
        copies.append(pltpu.async_copy(v_hbm.at[idx_k.at[isl]], rows_vk.at[sl], sem))
    for cp in copies:
        cp.wait()

    lanes = lax.iota(jnp.int32, L)

    def chunk(cidx, carry):
        rbase = cidx * L
        rows = rbase + lanes
        accj = jnp.zeros((L,), jnp.float32)
        acck = jnp.zeros((L,), jnp.float32)
        for d in range(D):
            col = (lanes + d) & (D - 1)
            u = plsc.load_gather(rows_u, [rows, col])
            vj = plsc.load_gather(rows_vj, [rows, col])
            vk = plsc.load_gather(rows_vk, [rows, col])
            dj = u - vj
            dk = u - vk
            accj = accj + dj * dj
            acck = acck + dk * dk
        x = acck - accj
        out_v[pl.ds(rbase, L)] = 1.0 / (1.0 + jnp.exp(-x))
        return carry

    lax.fori_loop(0, BPW // L, chunk, 0)
    pltpu.sync_copy(out_v, out_hbm.at[pl.ds(base, BPW)])


@jax.jit
def kernel(i, j, k, U, V):
    relayout = pl.kernel(
        _relayout_body,
        out_type=(jax.ShapeDtypeStruct((NM * D,), jnp.float32),
                  jax.ShapeDtypeStruct((NM * D,), jnp.float32)),
        mesh=plsc.VectorSubcoreMesh(**_MESH),
        scratch_types=[
            pltpu.VMEM((NBUF * D, MB), jnp.float32),
            pltpu.VMEM((NBUF, MB * D), jnp.float32),
            pltpu.SemaphoreType.DMA,
            pltpu.SemaphoreType.DMA,
        ],
        compiler_params=pltpu.CompilerParams(
            needs_layout_passes=False, use_tc_tiling_on_sc=True),
    )
    u_tail = U[NFULL * MB:, :].reshape(TAIL * D)
    v_tail = V[NFULL * MB:, :].reshape(TAIL * D)
    U_lin, V_lin = relayout(U.T, V.T, u_tail, v_tail)

    gather = pl.kernel(
        _gather_body,
        out_type=jax.ShapeDtypeStruct((B,), jnp.float32),
        mesh=plsc.VectorSubcoreMesh(**_MESH),
        scratch_types=[
            pltpu.VMEM((BPW,), jnp.int32),
            pltpu.VMEM((BPW,), jnp.int32),
            pltpu.VMEM((BPW,), jnp.int32),
            pltpu.VMEM((BPW, D), jnp.float32),
            pltpu.VMEM((BPW, D), jnp.float32),
            pltpu.VMEM((BPW, D), jnp.float32),
            pltpu.VMEM((BPW,), jnp.float32),
            pltpu.SemaphoreType.DMA,
        ],
        compiler_params=pltpu.CompilerParams(
            needs_layout_passes=False, use_tc_tiling_on_sc=False),
    )
    return gather(i, j, k, U_lin.reshape(NM, D), V_lin.reshape(NM, D))


# speedup vs baseline: 2.0126x; 1.2431x over previous
import jax
import jax.numpy as jnp
from jax import lax
from jax.experimental import pallas as pl
from jax.experimental.pallas import tpu as pltpu
from jax.experimental.pallas import tpu_sc as plsc

NC = 2
NS = 16
L = 16
NW = NC * NS
B = 16384
D = 32
NM = 100000
MB = 128
NFULL = NM // MB
TAIL = NM - NFULL * MB
BPW = B // NW
ICH = 128
NCH = BPW // ICH
NBUF = 4

_MESH = dict(core_axis_name="c", subcore_axis_name="s",
             num_cores=NC, num_subcores=NS)


def _relayout_body(ut_hbm, vt_hbm, utail_hbm, vtail_hbm, u_out, v_out,
                   blk2, stage2, in_sem, out_sem):
    c = lax.axis_index("c")
    s = lax.axis_index("s")
    wid = s * NC + c
    lanes = lax.iota(jnp.int32, L)

    def do_table(t_hbm, tail_hbm, t_out):
        nrounds = -(-NFULL // NW)

        def start_in(r):
            beta = r * NW + wid

            @pl.when(beta < NFULL)
            def _():
                pltpu.async_copy(t_hbm.at[:, pl.ds(beta * MB, MB)],
                                 blk2.at[pl.ds((r % NBUF) * D, D)], in_sem)

        for r0 in range(NBUF - 1):
            start_in(r0)

        def round_fn(r, carry):
            beta = r * NW + wid
            par = r % NBUF

            @pl.when(beta < NFULL)
            def _():
                m0 = beta * MB
                pltpu.make_async_copy(t_hbm.at[:, pl.ds(m0, MB)],
                                      blk2.at[pl.ds(par * D, D)],
                                      in_sem).wait()
                start_in(r + NBUF - 1)
                @pl.when(r >= NBUF)
                def _():
                    b2 = (r - NBUF) * NW + wid
                    pltpu.make_async_copy(
                        stage2.at[par],
                        t_out.at[pl.ds(b2 * MB * D, MB * D)],
                        out_sem).wait()
                parv = jnp.full((L,), par, jnp.int32)
                lanes32 = lanes * D

                @plsc.parallel_loop(0, MB, step=L, unroll=4)
                def _(rho0):
                    rho = rho0 + lanes
                    rho32 = rho0 * D + lanes32
                    for c0 in range(D):
                        cl = (c0 + lanes) & (D - 1)
                        val = plsc.load_gather(blk2, [par * D + cl, rho])
                        plsc.store_scatter(stage2, [parv, rho32 + cl], val)
                pltpu.async_copy(stage2.at[par],
                                 t_out.at[pl.ds(m0 * D, MB * D)],
                                 out_sem)
            return carry

        lax.fori_loop(0, nrounds, round_fn, 0)

        for r in range(max(nrounds - NBUF - 1, 0), nrounds):
            beta = r * NW + wid

            @pl.when((beta < NFULL) & (beta + NBUF * NW >= NFULL))
            def _(r=r, beta=beta):
                pltpu.make_async_copy(
                    stage2.at[r % NBUF],
                    t_out.at[pl.ds(beta * MB * D, MB * D)],
                    out_sem).wait()

        @pl.when(wid == 0)
        def _():
            pltpu.sync_copy(tail_hbm, stage2.at[0, pl.ds(0, TAIL * D)])
            pltpu.sync_copy(stage2.at[0, pl.ds(0, TAIL * D)],
                            t_out.at[pl.ds(NFULL * MB * D, TAIL * D)])

    do_table(ut_hbm, utail_hbm, u_out)
    do_table(vt_hbm, vtail_hbm, v_out)


def _gather_body(i_hbm, j_hbm, k_hbm, u_hbm, v_hbm, out_hbm,
                 idx_i, idx_j, idx_k, rows_u, rows_vj, rows_vk, out_v, sem):
    c = lax.axis_index("c")
    s = lax.axis_index("s")
    wid = s * NC + c
    base = wid * BPW

    pltpu.sync_copy(i_hbm.at[pl.ds(base, BPW)], idx_i)
    pltpu.sync_copy(j_hbm.at[pl.ds(base, BPW)], idx_j)
    pltpu.sync_copy(k_hbm.at[pl.ds(base, BPW)], idx_k)

    copies = []
    for q in range(NCH):
        isl = pl.ds(q * ICH, ICH)
        sl = pl.ds(q * ICH, ICH)
        copies.append(pltpu.async_copy(u_hbm.at[idx_i.at[isl]], rows_u.at[sl], sem))
        copies.append(pltpu.async_copy(v_hbm.at[idx_j.at[isl]], rows_vj.at[sl], sem))
        copies.append(pltpu.async_copy(v_hbm.at[idx_k.at[isl]], rows_vk.at[sl], sem))
    for cp in copies:
        cp.wait()

    lanes = lax.iota(jnp.int32, L)

    def chunk(cidx, carry):
        rbase = cidx * L
        rows = rbase + lanes
        accj = jnp.zeros((L,), jnp.float32)
        acck = jnp.zeros((L,), jnp.float32)
        for d in range(D):
            col = (lanes + d) & (D - 1)
            u = plsc.load_gather(rows_u, [rows, col])
            vj = plsc.load_gather(rows_vj, [rows, col])
            vk = plsc.load_gather(rows_vk, [rows, col])
            dj = u - vj
            dk = u - vk
            accj = accj + dj * dj
            acck = acck + dk * dk
        x = acck - accj
        out_v[pl.ds(rbase, L)] = 1.0 / (1.0 + jnp.exp(-x))
        return carry

    lax.fori_loop(0, BPW // L, chunk, 0)
    pltpu.sync_copy(out_v, out_hbm.at[pl.ds(base, BPW)])


@jax.jit
def kernel(i, j, k, U, V):
    relayout = pl.kernel(
        _relayout_body,
        out_type=(jax.ShapeDtypeStruct((NM * D,), jnp.float32),
                  jax.ShapeDtypeStruct((NM * D,), jnp.float32)),
        mesh=plsc.VectorSubcoreMesh(**_MESH),
        scratch_types=[
            pltpu.VMEM((NBUF * D, MB), jnp.float32),
            pltpu.VMEM((NBUF, MB * D), jnp.float32),
            pltpu.SemaphoreType.DMA,
            pltpu.SemaphoreType.DMA,
        ],
        compiler_params=pltpu.CompilerParams(
            needs_layout_passes=False, use_tc_tiling_on_sc=True),
    )
    u_tail = U[NFULL * MB:, :].reshape(TAIL * D)
    v_tail = V[NFULL * MB:, :].reshape(TAIL * D)
    U_lin, V_lin = relayout(U.T, V.T, u_tail, v_tail)

    gather = pl.kernel(
        _gather_body,
        out_type=jax.ShapeDtypeStruct((B,), jnp.float32),
        mesh=plsc.VectorSubcoreMesh(**_MESH),
        scratch_types=[
            pltpu.VMEM((BPW,), jnp.int32),
            pltpu.VMEM((BPW,), jnp.int32),
            pltpu.VMEM((BPW,), jnp.int32),
            pltpu.VMEM((BPW, D), jnp.float32),
            pltpu.VMEM((BPW, D), jnp.float32),
            pltpu.VMEM((BPW, D), jnp.float32),
            pltpu.VMEM((BPW,), jnp.float32),
            pltpu.SemaphoreType.DMA,
        ],
        compiler_params=pltpu.CompilerParams(
            needs_layout_passes=False, use_tc_tiling_on_sc=False),
    )
    return gather(i, j, k, U_lin.reshape(NM, D), V_lin.reshape(NM, D))
